# Initial kernel scaffold; baseline (speedup 1.0000x reference)
import jax
import jax.numpy as jnp
from jax.experimental import pallas as pl


def kernel(word_ids, table):
    raise NotImplementedError("write your pallas kernel here")


# speedup vs baseline: 1.0380x; 1.0380x over previous
import functools

import jax
import jax.numpy as jnp
from jax import lax
from jax.experimental import pallas as pl
from jax.experimental.pallas import tpu as pltpu
from jax.experimental.pallas import tpu_sc as plsc

B = 4096
L = 50
D = 64
NC = 2
NS = 16
NW = NC * NS
BPW = B // NW
LANES = 16
COLS = D // LANES


def _sc_body(idx_hbm, table_hbm, out_hbm, idx_v, acc_v, sem):
    c = lax.axis_index("c")
    s = lax.axis_index("s")
    wid = s * NC + c

    pltpu.sync_copy(idx_hbm.at[wid], idx_v)

    pltpu.async_copy(table_hbm.at[idx_v.at[0]], acc_v, sem).wait()

    def fire(l, carry):
        pltpu.async_copy(table_hbm.at[idx_v.at[l]], acc_v, sem, add=True)
        return carry

    lax.fori_loop(1, L, fire, 0)

    def drain(l, carry):
        pltpu.make_async_copy(table_hbm.at[idx_v.at[0]], acc_v, sem).wait()
        return carry

    lax.fori_loop(1, L, drain, 0)

    inv = jnp.float32(1.0 / L)

    def scale(i, carry):
        for cth in range(COLS):
            sl = pl.ds(cth * LANES, LANES)
            acc_v[i, sl] = acc_v[i, sl] * inv
        return carry

    lax.fori_loop(0, BPW, scale, 0)

    pltpu.sync_copy(acc_v, out_hbm.at[pl.ds(wid * BPW, BPW)])


_mesh = plsc.VectorSubcoreMesh(core_axis_name="c", subcore_axis_name="s")

_sc_call = functools.partial(
    pl.kernel,
    out_type=jax.ShapeDtypeStruct((B, D), jnp.float32),
    mesh=_mesh,
    scratch_types=[
        pltpu.VMEM((L, BPW), jnp.int32),
        pltpu.VMEM((BPW, D), jnp.float32),
        pltpu.SemaphoreType.DMA,
    ],
    compiler_params=pltpu.CompilerParams(use_tc_tiling_on_sc=False),
)(_sc_body)


def kernel(word_ids, table):
    idx = word_ids.astype(jnp.int32)
    idx3 = idx.T.reshape(L, NW, BPW).transpose(1, 0, 2)
    return _sc_call(idx3, table.astype(jnp.float32))
